# baseline (speedup 1.0000x reference)
import functools

import jax
import jax.numpy as jnp
from jax import lax
from jax.experimental import pallas as pl
from jax.experimental.pallas import tpu as pltpu
from jax.experimental.pallas import tpu_sc as plsc

N = 10000
E = 160000
D_IN = 256
D_HID = 32
N_CLASSES = 128
NUM_GRAPHS = 64

NW = 32
CHUNK = 128
NC_TOT = 80
NC0 = 32
NC1 = NC_TOT - NC0
NCMAX = 64
E_PAD = 16 * NC_TOT * CHUNK
NCHUNKS_ALL = E_PAD // CHUNK
ROWS_PER_TILE = 632
ACC_ROWS = 10112
TAIL_ROWS = N - 15 * ROWS_PER_TILE

NB = 10
BN = 1000

_mesh = plsc.VectorSubcoreMesh(core_axis_name="c", subcore_axis_name="s")


@functools.partial(
    pl.kernel,
    out_type=jax.ShapeDtypeStruct((2, N, 16), jnp.float32),
    mesh=_mesh,
    compiler_params=pltpu.CompilerParams(use_tc_tiling_on_sc=False),
    scratch_types=[
        pltpu.VMEM((NCMAX, CHUNK), jnp.int32),
        pltpu.VMEM((CHUNK, 16), jnp.float32),
        pltpu.VMEM((ROWS_PER_TILE, 16), jnp.float32),
        pltpu.VMEM_SHARED((ACC_ROWS, 16), jnp.float32),
        pltpu.SemaphoreType.DMA,
    ],
)
def _sc_degree(dst_hbm, out_hbm, dst_v, ones_v, zbuf, acc, ssem):
    c = lax.axis_index("c")
    s = lax.axis_index("s")

    z16 = jnp.zeros((16,), jnp.float32)
    o16 = jnp.ones((16,), jnp.float32)

    def _init(i, carry):
        zbuf[i, pl.ds(0, 16)] = z16
        return carry
    lax.fori_loop(0, ROWS_PER_TILE, _init, 0)

    def _ones(i, carry):
        ones_v[i, pl.ds(0, 16)] = o16
        return carry
    lax.fori_loop(0, CHUNK, _ones, 0)

    pltpu.sync_copy(zbuf, acc.at[pl.ds(s * ROWS_PER_TILE, ROWS_PER_TILE)])
    plsc.subcore_barrier()

    def _run(nc, base):
        pltpu.sync_copy(dst_hbm.at[pl.ds(base, nc)], dst_v.at[pl.ds(0, nc)])

        def _chunk(j, carry):
            pltpu.async_copy(ones_v, acc.at[dst_v.at[j]], ssem, add=True)
            return carry
        lax.fori_loop(0, nc, _chunk, 0)

        def _drain(j, carry):
            pltpu.make_async_copy(ones_v, acc.at[dst_v.at[0]], ssem).wait()
            return carry
        lax.fori_loop(0, nc, _drain, 0)

    @pl.when(c == 0)
    def _():
        _run(NC0, s * NC0)

    @pl.when(c == 1)
    def _():
        _run(NC1, 16 * NC0 + s * NC1)
    plsc.subcore_barrier()

    @pl.when(s < 15)
    def _():
        pltpu.sync_copy(acc.at[pl.ds(s * ROWS_PER_TILE, ROWS_PER_TILE)],
                        out_hbm.at[c, pl.ds(s * ROWS_PER_TILE, ROWS_PER_TILE)])

    @pl.when(s == 15)
    def _():
        pltpu.sync_copy(acc.at[pl.ds(15 * ROWS_PER_TILE, TAIL_ROWS)],
                        out_hbm.at[c, pl.ds(15 * ROWS_PER_TILE, TAIL_ROWS)])


NBUF = 4
LA = 2

@functools.partial(
    pl.kernel,
    out_type=jax.ShapeDtypeStruct((2, N, D_HID), jnp.bfloat16),
    mesh=_mesh,
    compiler_params=pltpu.CompilerParams(use_tc_tiling_on_sc=False),
    scratch_types=[
        pltpu.VMEM((NCMAX, CHUNK), jnp.int32),
        pltpu.VMEM((NCMAX, CHUNK), jnp.int32),
        pltpu.VMEM((NBUF, CHUNK, D_HID), jnp.bfloat16),
        pltpu.VMEM((ROWS_PER_TILE, D_HID), jnp.bfloat16),
        pltpu.VMEM_SHARED((ACC_ROWS, D_HID), jnp.bfloat16),
        [pltpu.SemaphoreType.DMA] * NBUF,
        [pltpu.SemaphoreType.DMA] * NBUF,
    ],
)
def _sc_aggregate(hs_hbm, src_hbm, dst_hbm, out_hbm,
                  src_v, dst_v, rows_v, zbuf, acc, gsems, ssems):
    c = lax.axis_index("c")
    s = lax.axis_index("s")

    z32 = jnp.zeros((32,), jnp.bfloat16)

    def _init(i, carry):
        zbuf[i, pl.ds(0, 32)] = z32
        return carry
    lax.fori_loop(0, ROWS_PER_TILE, _init, 0)

    pltpu.sync_copy(zbuf, acc.at[pl.ds(s * ROWS_PER_TILE, ROWS_PER_TILE)])
    plsc.subcore_barrier()

    def _gather(jj, b):
        pltpu.async_copy(hs_hbm.at[src_v.at[jj]], rows_v.at[b], gsems[b])

    def _gather_wait(jj, b):
        pltpu.make_async_copy(
            hs_hbm.at[src_v.at[jj]], rows_v.at[b], gsems[b]).wait()

    def _scatter(jj, b):
        pltpu.async_copy(rows_v.at[b], acc.at[dst_v.at[jj]], ssems[b],
                         add=True)

    def _scatter_wait(b):
        pltpu.make_async_copy(rows_v.at[b], acc.at[dst_v.at[0]],
                              ssems[b]).wait()

    def _run(nc, base):
        pltpu.sync_copy(src_hbm.at[pl.ds(base, nc)], src_v.at[pl.ds(0, nc)])
        pltpu.sync_copy(dst_hbm.at[pl.ds(base, nc)], dst_v.at[pl.ds(0, nc)])

        for jj in range(LA):
            _gather(jj, jj % NBUF)

        def _outer(k, carry):
            for b0 in range(NBUF):
                jj = k * NBUF + b0
                bl = (b0 + LA) % NBUF

                @pl.when(jj >= LA)
                def _():
                    _scatter_wait(bl)

                @pl.when(jj + LA < nc)
                def _():
                    _gather(jj + LA, bl)

                _gather_wait(jj, b0)
                _scatter(jj, b0)
            return carry
        lax.fori_loop(0, nc // NBUF, _outer, 0)

        for b in range((nc - LA) % NBUF, (nc - LA) % NBUF + LA):
            _scatter_wait(b % NBUF)

    @pl.when(c == 0)
    def _():
        _run(NC0, s * NC0)

    @pl.when(c == 1)
    def _():
        _run(NC1, 16 * NC0 + s * NC1)
    plsc.subcore_barrier()

    @pl.when(s < 15)
    def _():
        pltpu.sync_copy(acc.at[pl.ds(s * ROWS_PER_TILE, ROWS_PER_TILE)],
                        out_hbm.at[c, pl.ds(s * ROWS_PER_TILE, ROWS_PER_TILE)])

    @pl.when(s == 15)
    def _():
        pltpu.sync_copy(acc.at[pl.ds(15 * ROWS_PER_TILE, TAIL_ROWS)],
                        out_hbm.at[c, pl.ds(15 * ROWS_PER_TILE, TAIL_ROWS)])


def _mm_scale_body(cnt_ref, x_ref, w_ref, hs_ref, dis_ref):
    p = jnp.dot(x_ref[...], w_ref[...], preferred_element_type=jnp.float32)
    deg = cnt_ref[0, :, 0:1] + cnt_ref[1, :, 0:1] + 1.0
    dis = lax.rsqrt(deg)
    dis_ref[...] = dis
    hs_ref[...] = (p * dis).astype(jnp.bfloat16)


def _tc_mm_scale(cnt, x, W1):
    return pl.pallas_call(
        _mm_scale_body,
        grid=(NB,),
        in_specs=[pl.BlockSpec((2, BN, 16), lambda i: (0, i, 0)),
                  pl.BlockSpec((BN, D_IN), lambda i: (i, 0)),
                  pl.BlockSpec((D_IN, D_HID), lambda i: (0, 0))],
        out_specs=[pl.BlockSpec((BN, D_HID), lambda i: (i, 0)),
                   pl.BlockSpec((BN, 1), lambda i: (i, 0))],
        out_shape=[jax.ShapeDtypeStruct((N, D_HID), jnp.bfloat16),
                   jax.ShapeDtypeStruct((N, 1), jnp.float32)],
    )(cnt, x, W1)


def _l1_body(a_ref, hs_ref, dis_ref, b_ref, o_ref):
    ssum = (a_ref[0].astype(jnp.float32) + a_ref[1].astype(jnp.float32)
            + hs_ref[...].astype(jnp.float32))
    h1 = jnp.maximum(ssum * dis_ref[...] + b_ref[...], 0.0)
    o_ref[...] = (h1 * dis_ref[...]).astype(jnp.bfloat16)


def _tc_l1(agg1, hs1, dis, b1r):
    return pl.pallas_call(
        _l1_body,
        grid=(NB,),
        in_specs=[pl.BlockSpec((2, BN, D_HID), lambda i: (0, i, 0)),
                  pl.BlockSpec((BN, D_HID), lambda i: (i, 0)),
                  pl.BlockSpec((BN, 1), lambda i: (i, 0)),
                  pl.BlockSpec((1, D_HID), lambda i: (0, 0))],
        out_specs=pl.BlockSpec((BN, D_HID), lambda i: (i, 0)),
        out_shape=jax.ShapeDtypeStruct((N, D_HID), jnp.bfloat16),
    )(agg1, hs1, dis, b1r)


def _final_body(a_ref, hs_ref, dis_ref, batch_ref, w_ref, b_ref, o_ref,
                sums, cnts):
    i = pl.program_id(0)

    @pl.when(i == 0)
    def _():
        sums[...] = jnp.zeros_like(sums)
        cnts[...] = jnp.zeros_like(cnts)

    m = (a_ref[0].astype(jnp.float32) + a_ref[1].astype(jnp.float32)
         + hs_ref[...].astype(jnp.float32)) * dis_ref[...]
    h2 = jnp.maximum(
        jnp.dot(m, w_ref[...], preferred_element_type=jnp.float32)
        + b_ref[...], 0.0)
    g_iota = lax.broadcasted_iota(jnp.int32, (NUM_GRAPHS, BN), 0)
    bvec = jnp.broadcast_to(batch_ref[0], (NUM_GRAPHS, BN))
    maskf = (g_iota == bvec).astype(jnp.float32)
    sums[...] += jnp.dot(maskf, h2, preferred_element_type=jnp.float32)
    cnts[...] += jnp.broadcast_to(
        jnp.sum(maskf, axis=1, keepdims=True), (NUM_GRAPHS, N_CLASSES))

    @pl.when(i == NB - 1)
    def _():
        pooled = sums[...] / jnp.maximum(cnts[...], 1.0)
        z = pooled - jnp.max(pooled, axis=1, keepdims=True)
        lse = jnp.log(jnp.sum(jnp.exp(z), axis=1, keepdims=True))
        o_ref[...] = z - lse


def _tc_final(agg2, hs2, dis, batch3, W2, b2r):
    return pl.pallas_call(
        _final_body,
        grid=(NB,),
        in_specs=[pl.BlockSpec((2, BN, D_HID), lambda i: (0, i, 0)),
                  pl.BlockSpec((BN, D_HID), lambda i: (i, 0)),
                  pl.BlockSpec((BN, 1), lambda i: (i, 0)),
                  pl.BlockSpec((1, 1, BN), lambda i: (i, 0, 0)),
                  pl.BlockSpec((D_HID, N_CLASSES), lambda i: (0, 0)),
                  pl.BlockSpec((1, N_CLASSES), lambda i: (0, 0))],
        out_specs=pl.BlockSpec((NUM_GRAPHS, N_CLASSES), lambda i: (0, 0)),
        out_shape=jax.ShapeDtypeStruct((NUM_GRAPHS, N_CLASSES), jnp.float32),
        scratch_shapes=[pltpu.VMEM((NUM_GRAPHS, N_CLASSES), jnp.float32),
                        pltpu.VMEM((NUM_GRAPHS, N_CLASSES), jnp.float32)],
    )(agg2, hs2, dis, batch3, W2, b2r)


def kernel(x, edge_index, batch, W1, b1, W2, b2):
    src = edge_index[0]
    dst = edge_index[1]
    pad = E_PAD - E
    src_p = jnp.concatenate(
        [src, jnp.zeros((pad,), jnp.int32)]).reshape(NCHUNKS_ALL, CHUNK)
    dst_p = jnp.concatenate(
        [dst, jnp.full((pad,), N, jnp.int32)]).reshape(NCHUNKS_ALL, CHUNK)
    batch3 = batch.reshape(NB, 1, BN)
    b1r = b1.reshape(1, D_HID)
    b2r = b2.reshape(1, N_CLASSES)

    cnt = _sc_degree(dst_p)
    hs1, dis = _tc_mm_scale(cnt, x, W1)
    agg1 = _sc_aggregate(hs1, src_p, dst_p)
    hs2 = _tc_l1(agg1, hs1, dis, b1r)
    agg2 = _sc_aggregate(hs2, src_p, dst_p)
    return _tc_final(agg2, hs2, dis, batch3, W2, b2r)


# speedup vs baseline: 1.0615x; 1.0615x over previous
import functools

import jax
import jax.numpy as jnp
from jax import lax
from jax.experimental import pallas as pl
from jax.experimental.pallas import tpu as pltpu
from jax.experimental.pallas import tpu_sc as plsc

N = 10000
E = 160000
D_IN = 256
D_HID = 32
N_CLASSES = 128
NUM_GRAPHS = 64

NW = 32
CHUNK = 128
NC_TOT = 80
NC0 = 48
NC1 = NC_TOT - NC0
NCMAX = 64
E_PAD = 16 * NC_TOT * CHUNK
NCHUNKS_ALL = E_PAD // CHUNK
ROWS_PER_TILE = 632
ACC_ROWS = 10112
TAIL_ROWS = N - 15 * ROWS_PER_TILE

NB = 10
BN = 1000

_mesh = plsc.VectorSubcoreMesh(core_axis_name="c", subcore_axis_name="s")


@functools.partial(
    pl.kernel,
    out_type=jax.ShapeDtypeStruct((2, N, 16), jnp.float32),
    mesh=_mesh,
    compiler_params=pltpu.CompilerParams(use_tc_tiling_on_sc=False),
    scratch_types=[
        pltpu.VMEM((NCMAX, CHUNK), jnp.int32),
        pltpu.VMEM((CHUNK, 16), jnp.float32),
        pltpu.VMEM((ROWS_PER_TILE, 16), jnp.float32),
        pltpu.VMEM_SHARED((ACC_ROWS, 16), jnp.float32),
        pltpu.SemaphoreType.DMA,
    ],
)
def _sc_degree(dst_hbm, out_hbm, dst_v, ones_v, zbuf, acc, ssem):
    c = lax.axis_index("c")
    s = lax.axis_index("s")

    z16 = jnp.zeros((16,), jnp.float32)
    o16 = jnp.ones((16,), jnp.float32)

    def _init(i, carry):
        zbuf[i, pl.ds(0, 16)] = z16
        return carry
    lax.fori_loop(0, ROWS_PER_TILE, _init, 0)

    def _ones(i, carry):
        ones_v[i, pl.ds(0, 16)] = o16
        return carry
    lax.fori_loop(0, CHUNK, _ones, 0)

    pltpu.sync_copy(zbuf, acc.at[pl.ds(s * ROWS_PER_TILE, ROWS_PER_TILE)])
    plsc.subcore_barrier()

    def _run(nc, base):
        pltpu.sync_copy(dst_hbm.at[pl.ds(base, nc)], dst_v.at[pl.ds(0, nc)])

        def _chunk(j, carry):
            pltpu.async_copy(ones_v, acc.at[dst_v.at[j]], ssem, add=True)
            return carry
        lax.fori_loop(0, nc, _chunk, 0)

        def _drain(j, carry):
            pltpu.make_async_copy(ones_v, acc.at[dst_v.at[0]], ssem).wait()
            return carry
        lax.fori_loop(0, nc, _drain, 0)

    @pl.when(c == 0)
    def _():
        _run(NC0, s * NC0)

    @pl.when(c == 1)
    def _():
        _run(NC1, 16 * NC0 + s * NC1)
    plsc.subcore_barrier()

    @pl.when(s < 15)
    def _():
        pltpu.sync_copy(acc.at[pl.ds(s * ROWS_PER_TILE, ROWS_PER_TILE)],
                        out_hbm.at[c, pl.ds(s * ROWS_PER_TILE, ROWS_PER_TILE)])

    @pl.when(s == 15)
    def _():
        pltpu.sync_copy(acc.at[pl.ds(15 * ROWS_PER_TILE, TAIL_ROWS)],
                        out_hbm.at[c, pl.ds(15 * ROWS_PER_TILE, TAIL_ROWS)])


NBUF = 4
LA = 2

@functools.partial(
    pl.kernel,
    out_type=jax.ShapeDtypeStruct((2, N, D_HID), jnp.bfloat16),
    mesh=_mesh,
    compiler_params=pltpu.CompilerParams(use_tc_tiling_on_sc=False),
    scratch_types=[
        pltpu.VMEM((NCMAX, CHUNK), jnp.int32),
        pltpu.VMEM((NCMAX, CHUNK), jnp.int32),
        pltpu.VMEM((NBUF, CHUNK, D_HID), jnp.bfloat16),
        pltpu.VMEM((ROWS_PER_TILE, D_HID), jnp.bfloat16),
        pltpu.VMEM_SHARED((ACC_ROWS, D_HID), jnp.bfloat16),
        [pltpu.SemaphoreType.DMA] * NBUF,
        [pltpu.SemaphoreType.DMA] * NBUF,
    ],
)
def _sc_aggregate(hs_hbm, src_hbm, dst_hbm, out_hbm,
                  src_v, dst_v, rows_v, zbuf, acc, gsems, ssems):
    c = lax.axis_index("c")
    s = lax.axis_index("s")

    z32 = jnp.zeros((32,), jnp.bfloat16)

    def _init(i, carry):
        zbuf[i, pl.ds(0, 32)] = z32
        return carry
    lax.fori_loop(0, ROWS_PER_TILE, _init, 0)

    pltpu.sync_copy(zbuf, acc.at[pl.ds(s * ROWS_PER_TILE, ROWS_PER_TILE)])
    plsc.subcore_barrier()

    def _gather(jj, b):
        pltpu.async_copy(hs_hbm.at[src_v.at[jj]], rows_v.at[b], gsems[b])

    def _gather_wait(jj, b):
        pltpu.make_async_copy(
            hs_hbm.at[src_v.at[jj]], rows_v.at[b], gsems[b]).wait()

    def _scatter(jj, b):
        pltpu.async_copy(rows_v.at[b], acc.at[dst_v.at[jj]], ssems[b],
                         add=True)

    def _scatter_wait(b):
        pltpu.make_async_copy(rows_v.at[b], acc.at[dst_v.at[0]],
                              ssems[b]).wait()

    def _run(nc, base):
        pltpu.sync_copy(src_hbm.at[pl.ds(base, nc)], src_v.at[pl.ds(0, nc)])
        pltpu.sync_copy(dst_hbm.at[pl.ds(base, nc)], dst_v.at[pl.ds(0, nc)])

        for jj in range(LA):
            _gather(jj, jj % NBUF)

        def _outer(k, carry):
            for b0 in range(NBUF):
                jj = k * NBUF + b0
                bl = (b0 + LA) % NBUF

                @pl.when(jj >= LA)
                def _():
                    _scatter_wait(bl)

                @pl.when(jj + LA < nc)
                def _():
                    _gather(jj + LA, bl)

                _gather_wait(jj, b0)
                _scatter(jj, b0)
            return carry
        lax.fori_loop(0, nc // NBUF, _outer, 0)

        for b in range((nc - LA) % NBUF, (nc - LA) % NBUF + LA):
            _scatter_wait(b % NBUF)

    @pl.when(c == 0)
    def _():
        _run(NC0, s * NC0)

    @pl.when(c == 1)
    def _():
        _run(NC1, 16 * NC0 + s * NC1)
    plsc.subcore_barrier()

    @pl.when(s < 15)
    def _():
        pltpu.sync_copy(acc.at[pl.ds(s * ROWS_PER_TILE, ROWS_PER_TILE)],
                        out_hbm.at[c, pl.ds(s * ROWS_PER_TILE, ROWS_PER_TILE)])

    @pl.when(s == 15)
    def _():
        pltpu.sync_copy(acc.at[pl.ds(15 * ROWS_PER_TILE, TAIL_ROWS)],
                        out_hbm.at[c, pl.ds(15 * ROWS_PER_TILE, TAIL_ROWS)])


def _mm_scale_body(cnt_ref, x_ref, w_ref, hs_ref, dis_ref):
    p = jnp.dot(x_ref[...], w_ref[...], preferred_element_type=jnp.float32)
    deg = cnt_ref[0, :, 0:1] + cnt_ref[1, :, 0:1] + 1.0
    dis = lax.rsqrt(deg)
    dis_ref[...] = dis
    hs_ref[...] = (p * dis).astype(jnp.bfloat16)


def _tc_mm_scale(cnt, x, W1):
    return pl.pallas_call(
        _mm_scale_body,
        grid=(NB,),
        in_specs=[pl.BlockSpec((2, BN, 16), lambda i: (0, i, 0)),
                  pl.BlockSpec((BN, D_IN), lambda i: (i, 0)),
                  pl.BlockSpec((D_IN, D_HID), lambda i: (0, 0))],
        out_specs=[pl.BlockSpec((BN, D_HID), lambda i: (i, 0)),
                   pl.BlockSpec((BN, 1), lambda i: (i, 0))],
        out_shape=[jax.ShapeDtypeStruct((N, D_HID), jnp.bfloat16),
                   jax.ShapeDtypeStruct((N, 1), jnp.float32)],
    )(cnt, x, W1)


def _l1_body(a_ref, hs_ref, dis_ref, b_ref, o_ref):
    ssum = (a_ref[0].astype(jnp.float32) + a_ref[1].astype(jnp.float32)
            + hs_ref[...].astype(jnp.float32))
    h1 = jnp.maximum(ssum * dis_ref[...] + b_ref[...], 0.0)
    o_ref[...] = (h1 * dis_ref[...]).astype(jnp.bfloat16)


def _tc_l1(agg1, hs1, dis, b1r):
    return pl.pallas_call(
        _l1_body,
        grid=(NB,),
        in_specs=[pl.BlockSpec((2, BN, D_HID), lambda i: (0, i, 0)),
                  pl.BlockSpec((BN, D_HID), lambda i: (i, 0)),
                  pl.BlockSpec((BN, 1), lambda i: (i, 0)),
                  pl.BlockSpec((1, D_HID), lambda i: (0, 0))],
        out_specs=pl.BlockSpec((BN, D_HID), lambda i: (i, 0)),
        out_shape=jax.ShapeDtypeStruct((N, D_HID), jnp.bfloat16),
    )(agg1, hs1, dis, b1r)


def _final_body(a_ref, hs_ref, dis_ref, batch_ref, w_ref, b_ref, o_ref,
                sums, cnts):
    i = pl.program_id(0)

    @pl.when(i == 0)
    def _():
        sums[...] = jnp.zeros_like(sums)
        cnts[...] = jnp.zeros_like(cnts)

    m = (a_ref[0].astype(jnp.float32) + a_ref[1].astype(jnp.float32)
         + hs_ref[...].astype(jnp.float32)) * dis_ref[...]
    h2 = jnp.maximum(
        jnp.dot(m, w_ref[...], preferred_element_type=jnp.float32)
        + b_ref[...], 0.0)
    g_iota = lax.broadcasted_iota(jnp.int32, (NUM_GRAPHS, BN), 0)
    bvec = jnp.broadcast_to(batch_ref[0], (NUM_GRAPHS, BN))
    maskf = (g_iota == bvec).astype(jnp.float32)
    sums[...] += jnp.dot(maskf, h2, preferred_element_type=jnp.float32)
    cnts[...] += jnp.broadcast_to(
        jnp.sum(maskf, axis=1, keepdims=True), (NUM_GRAPHS, N_CLASSES))

    @pl.when(i == NB - 1)
    def _():
        pooled = sums[...] / jnp.maximum(cnts[...], 1.0)
        z = pooled - jnp.max(pooled, axis=1, keepdims=True)
        lse = jnp.log(jnp.sum(jnp.exp(z), axis=1, keepdims=True))
        o_ref[...] = z - lse


def _tc_final(agg2, hs2, dis, batch3, W2, b2r):
    return pl.pallas_call(
        _final_body,
        grid=(NB,),
        in_specs=[pl.BlockSpec((2, BN, D_HID), lambda i: (0, i, 0)),
                  pl.BlockSpec((BN, D_HID), lambda i: (i, 0)),
                  pl.BlockSpec((BN, 1), lambda i: (i, 0)),
                  pl.BlockSpec((1, 1, BN), lambda i: (i, 0, 0)),
                  pl.BlockSpec((D_HID, N_CLASSES), lambda i: (0, 0)),
                  pl.BlockSpec((1, N_CLASSES), lambda i: (0, 0))],
        out_specs=pl.BlockSpec((NUM_GRAPHS, N_CLASSES), lambda i: (0, 0)),
        out_shape=jax.ShapeDtypeStruct((NUM_GRAPHS, N_CLASSES), jnp.float32),
        scratch_shapes=[pltpu.VMEM((NUM_GRAPHS, N_CLASSES), jnp.float32),
                        pltpu.VMEM((NUM_GRAPHS, N_CLASSES), jnp.float32)],
    )(agg2, hs2, dis, batch3, W2, b2r)


def kernel(x, edge_index, batch, W1, b1, W2, b2):
    src = edge_index[0]
    dst = edge_index[1]
    pad = E_PAD - E
    src_p = jnp.concatenate(
        [src, jnp.zeros((pad,), jnp.int32)]).reshape(NCHUNKS_ALL, CHUNK)
    dst_p = jnp.concatenate(
        [dst, jnp.full((pad,), N, jnp.int32)]).reshape(NCHUNKS_ALL, CHUNK)
    batch3 = batch.reshape(NB, 1, BN)
    b1r = b1.reshape(1, D_HID)
    b2r = b2.reshape(1, N_CLASSES)

    cnt = _sc_degree(dst_p)
    hs1, dis = _tc_mm_scale(cnt, x, W1)
    agg1 = _sc_aggregate(hs1, src_p, dst_p)
    hs2 = _tc_l1(agg1, hs1, dis, b1r)
    agg2 = _sc_aggregate(hs2, src_p, dst_p)
    return _tc_final(agg2, hs2, dis, batch3, W2, b2r)


# speedup vs baseline: 1.0970x; 1.0334x over previous
import functools

import jax
import jax.numpy as jnp
from jax import lax
from jax.experimental import pallas as pl
from jax.experimental.pallas import tpu as pltpu
from jax.experimental.pallas import tpu_sc as plsc

N = 10000
E = 160000
D_IN = 256
D_HID = 32
N_CLASSES = 128
NUM_GRAPHS = 64

NW = 32
CHUNK = 128
NC_TOT = 80
NC0 = 48
NC1 = NC_TOT - NC0
NCMAX = 64
E_PAD = 16 * NC_TOT * CHUNK
NCHUNKS_ALL = E_PAD // CHUNK
ROWS_PER_TILE = 632
ACC_ROWS = 10112
TAIL_ROWS = N - 15 * ROWS_PER_TILE

NB = 10
BN = 1000

_mesh = plsc.VectorSubcoreMesh(core_axis_name="c", subcore_axis_name="s")


@functools.partial(
    pl.kernel,
    out_type=jax.ShapeDtypeStruct((2, N, 16), jnp.bfloat16),
    mesh=_mesh,
    compiler_params=pltpu.CompilerParams(use_tc_tiling_on_sc=False),
    scratch_types=[
        pltpu.VMEM((NCMAX, CHUNK), jnp.int32),
        pltpu.VMEM((CHUNK, 16), jnp.bfloat16),
        pltpu.VMEM((ROWS_PER_TILE, 16), jnp.bfloat16),
        pltpu.VMEM_SHARED((ACC_ROWS, 16), jnp.bfloat16),
        pltpu.SemaphoreType.DMA,
    ],
)
def _sc_degree(dst_hbm, out_hbm, dst_v, ones_v, zbuf, acc, ssem):
    c = lax.axis_index("c")
    s = lax.axis_index("s")

    z216 = jnp.zeros((2, 16), jnp.bfloat16)
    o216 = jnp.ones((2, 16), jnp.bfloat16)

    def _init(i, carry):
        zbuf[pl.ds(2 * i, 2), :] = z216
        return carry
    lax.fori_loop(0, ROWS_PER_TILE // 2, _init, 0)

    def _ones(i, carry):
        ones_v[pl.ds(2 * i, 2), :] = o216
        return carry
    lax.fori_loop(0, CHUNK // 2, _ones, 0)

    pltpu.sync_copy(zbuf, acc.at[pl.ds(s * ROWS_PER_TILE, ROWS_PER_TILE)])
    plsc.subcore_barrier()

    def _run(nc, base):
        pltpu.sync_copy(dst_hbm.at[pl.ds(base, nc)], dst_v.at[pl.ds(0, nc)])

        def _chunk(j, carry):
            pltpu.async_copy(ones_v, acc.at[dst_v.at[j]], ssem, add=True)
            return carry
        lax.fori_loop(0, nc, _chunk, 0)

        def _drain(j, carry):
            pltpu.make_async_copy(ones_v, acc.at[dst_v.at[0]], ssem).wait()
            return carry
        lax.fori_loop(0, nc, _drain, 0)

    @pl.when(c == 0)
    def _():
        _run(NC0, s * NC0)

    @pl.when(c == 1)
    def _():
        _run(NC1, 16 * NC0 + s * NC1)
    plsc.subcore_barrier()

    @pl.when(s < 15)
    def _():
        pltpu.sync_copy(acc.at[pl.ds(s * ROWS_PER_TILE, ROWS_PER_TILE)],
                        out_hbm.at[c, pl.ds(s * ROWS_PER_TILE, ROWS_PER_TILE)])

    @pl.when(s == 15)
    def _():
        pltpu.sync_copy(acc.at[pl.ds(15 * ROWS_PER_TILE, TAIL_ROWS)],
                        out_hbm.at[c, pl.ds(15 * ROWS_PER_TILE, TAIL_ROWS)])


NBUF = 4
LA = 2

@functools.partial(
    pl.kernel,
    out_type=jax.ShapeDtypeStruct((2, N, D_HID), jnp.bfloat16),
    mesh=_mesh,
    compiler_params=pltpu.CompilerParams(use_tc_tiling_on_sc=False),
    scratch_types=[
        pltpu.VMEM((NCMAX, CHUNK), jnp.int32),
        pltpu.VMEM((NCMAX, CHUNK), jnp.int32),
        pltpu.VMEM((NBUF, CHUNK, D_HID), jnp.bfloat16),
        pltpu.VMEM((ROWS_PER_TILE, D_HID), jnp.bfloat16),
        pltpu.VMEM_SHARED((ACC_ROWS, D_HID), jnp.bfloat16),
        [pltpu.SemaphoreType.DMA] * NBUF,
        [pltpu.SemaphoreType.DMA] * NBUF,
    ],
)
def _sc_aggregate(hs_hbm, src_hbm, dst_hbm, out_hbm,
                  src_v, dst_v, rows_v, zbuf, acc, gsems, ssems):
    c = lax.axis_index("c")
    s = lax.axis_index("s")

    z32 = jnp.zeros((32,), jnp.bfloat16)

    def _init(i, carry):
        zbuf[i, pl.ds(0, 32)] = z32
        return carry
    lax.fori_loop(0, ROWS_PER_TILE, _init, 0)

    pltpu.sync_copy(zbuf, acc.at[pl.ds(s * ROWS_PER_TILE, ROWS_PER_TILE)])
    plsc.subcore_barrier()

    def _gather(jj, b):
        pltpu.async_copy(hs_hbm.at[src_v.at[jj]], rows_v.at[b], gsems[b])

    def _gather_wait(jj, b):
        pltpu.make_async_copy(
            hs_hbm.at[src_v.at[jj]], rows_v.at[b], gsems[b]).wait()

    def _scatter(jj, b):
        pltpu.async_copy(rows_v.at[b], acc.at[dst_v.at[jj]], ssems[b],
                         add=True)

    def _scatter_wait(b):
        pltpu.make_async_copy(rows_v.at[b], acc.at[dst_v.at[0]],
                              ssems[b]).wait()

    def _run(nc, base):
        pltpu.sync_copy(src_hbm.at[pl.ds(base, nc)], src_v.at[pl.ds(0, nc)])
        pltpu.sync_copy(dst_hbm.at[pl.ds(base, nc)], dst_v.at[pl.ds(0, nc)])

        for jj in range(LA):
            _gather(jj, jj % NBUF)

        def _outer(k, carry):
            for b0 in range(NBUF):
                jj = k * NBUF + b0
                bl = (b0 + LA) % NBUF

                @pl.when(jj >= LA)
                def _():
                    _scatter_wait(bl)

                @pl.when(jj + LA < nc)
                def _():
                    _gather(jj + LA, bl)

                _gather_wait(jj, b0)
                _scatter(jj, b0)
            return carry
        lax.fori_loop(0, nc // NBUF, _outer, 0)

        for b in range((nc - LA) % NBUF, (nc - LA) % NBUF + LA):
            _scatter_wait(b % NBUF)

    @pl.when(c == 0)
    def _():
        _run(NC0, s * NC0)

    @pl.when(c == 1)
    def _():
        _run(NC1, 16 * NC0 + s * NC1)
    plsc.subcore_barrier()

    @pl.when(s < 15)
    def _():
        pltpu.sync_copy(acc.at[pl.ds(s * ROWS_PER_TILE, ROWS_PER_TILE)],
                        out_hbm.at[c, pl.ds(s * ROWS_PER_TILE, ROWS_PER_TILE)])

    @pl.when(s == 15)
    def _():
        pltpu.sync_copy(acc.at[pl.ds(15 * ROWS_PER_TILE, TAIL_ROWS)],
                        out_hbm.at[c, pl.ds(15 * ROWS_PER_TILE, TAIL_ROWS)])


def _mm_scale_body(cnt_ref, x_ref, w_ref, hs_ref, dis_ref):
    p = jnp.dot(x_ref[...], w_ref[...], preferred_element_type=jnp.float32)
    deg = (cnt_ref[0, :, 0:1].astype(jnp.float32)
           + cnt_ref[1, :, 0:1].astype(jnp.float32) + 1.0)
    dis = lax.rsqrt(deg)
    dis_ref[...] = dis
    hs_ref[...] = (p * dis).astype(jnp.bfloat16)


def _tc_mm_scale(cnt, x, W1):
    return pl.pallas_call(
        _mm_scale_body,
        grid=(NB,),
        in_specs=[pl.BlockSpec((2, BN, 16), lambda i: (0, i, 0)),
                  pl.BlockSpec((BN, D_IN), lambda i: (i, 0)),
                  pl.BlockSpec((D_IN, D_HID), lambda i: (0, 0))],
        out_specs=[pl.BlockSpec((BN, D_HID), lambda i: (i, 0)),
                   pl.BlockSpec((BN, 1), lambda i: (i, 0))],
        out_shape=[jax.ShapeDtypeStruct((N, D_HID), jnp.bfloat16),
                   jax.ShapeDtypeStruct((N, 1), jnp.float32)],
    )(cnt, x, W1)


def _l1_body(a_ref, hs_ref, dis_ref, b_ref, o_ref):
    ssum = (a_ref[0].astype(jnp.float32) + a_ref[1].astype(jnp.float32)
            + hs_ref[...].astype(jnp.float32))
    h1 = jnp.maximum(ssum * dis_ref[...] + b_ref[...], 0.0)
    o_ref[...] = (h1 * dis_ref[...]).astype(jnp.bfloat16)


def _tc_l1(agg1, hs1, dis, b1r):
    return pl.pallas_call(
        _l1_body,
        grid=(NB,),
        in_specs=[pl.BlockSpec((2, BN, D_HID), lambda i: (0, i, 0)),
                  pl.BlockSpec((BN, D_HID), lambda i: (i, 0)),
                  pl.BlockSpec((BN, 1), lambda i: (i, 0)),
                  pl.BlockSpec((1, D_HID), lambda i: (0, 0))],
        out_specs=pl.BlockSpec((BN, D_HID), lambda i: (i, 0)),
        out_shape=jax.ShapeDtypeStruct((N, D_HID), jnp.bfloat16),
    )(agg1, hs1, dis, b1r)


def _final_body(a_ref, hs_ref, dis_ref, batch_ref, w_ref, b_ref, o_ref,
                sums, cnts):
    i = pl.program_id(0)

    @pl.when(i == 0)
    def _():
        sums[...] = jnp.zeros_like(sums)
        cnts[...] = jnp.zeros_like(cnts)

    m = (a_ref[0].astype(jnp.float32) + a_ref[1].astype(jnp.float32)
         + hs_ref[...].astype(jnp.float32)) * dis_ref[...]
    h2 = jnp.maximum(
        jnp.dot(m, w_ref[...], preferred_element_type=jnp.float32)
        + b_ref[...], 0.0)
    g_iota = lax.broadcasted_iota(jnp.int32, (NUM_GRAPHS, BN), 0)
    bvec = jnp.broadcast_to(batch_ref[0], (NUM_GRAPHS, BN))
    maskf = (g_iota == bvec).astype(jnp.float32)
    sums[...] += jnp.dot(maskf, h2, preferred_element_type=jnp.float32)
    cnts[...] += jnp.broadcast_to(
        jnp.sum(maskf, axis=1, keepdims=True), (NUM_GRAPHS, N_CLASSES))

    @pl.when(i == NB - 1)
    def _():
        pooled = sums[...] / jnp.maximum(cnts[...], 1.0)
        z = pooled - jnp.max(pooled, axis=1, keepdims=True)
        lse = jnp.log(jnp.sum(jnp.exp(z), axis=1, keepdims=True))
        o_ref[...] = z - lse


def _tc_final(agg2, hs2, dis, batch3, W2, b2r):
    return pl.pallas_call(
        _final_body,
        grid=(NB,),
        in_specs=[pl.BlockSpec((2, BN, D_HID), lambda i: (0, i, 0)),
                  pl.BlockSpec((BN, D_HID), lambda i: (i, 0)),
                  pl.BlockSpec((BN, 1), lambda i: (i, 0)),
                  pl.BlockSpec((1, 1, BN), lambda i: (i, 0, 0)),
                  pl.BlockSpec((D_HID, N_CLASSES), lambda i: (0, 0)),
                  pl.BlockSpec((1, N_CLASSES), lambda i: (0, 0))],
        out_specs=pl.BlockSpec((NUM_GRAPHS, N_CLASSES), lambda i: (0, 0)),
        out_shape=jax.ShapeDtypeStruct((NUM_GRAPHS, N_CLASSES), jnp.float32),
        scratch_shapes=[pltpu.VMEM((NUM_GRAPHS, N_CLASSES), jnp.float32),
                        pltpu.VMEM((NUM_GRAPHS, N_CLASSES), jnp.float32)],
    )(agg2, hs2, dis, batch3, W2, b2r)


def kernel(x, edge_index, batch, W1, b1, W2, b2):
    src = edge_index[0]
    dst = edge_index[1]
    pad = E_PAD - E
    src_p = jnp.concatenate(
        [src, jnp.zeros((pad,), jnp.int32)]).reshape(NCHUNKS_ALL, CHUNK)
    dst_p = jnp.concatenate(
        [dst, jnp.full((pad,), N, jnp.int32)]).reshape(NCHUNKS_ALL, CHUNK)
    batch3 = batch.reshape(NB, 1, BN)
    b1r = b1.reshape(1, D_HID)
    b2r = b2.reshape(1, N_CLASSES)

    cnt = _sc_degree(dst_p)
    hs1, dis = _tc_mm_scale(cnt, x, W1)
    agg1 = _sc_aggregate(hs1, src_p, dst_p)
    hs2 = _tc_l1(agg1, hs1, dis, b1r)
    agg2 = _sc_aggregate(hs2, src_p, dst_p)
    return _tc_final(agg2, hs2, dis, batch3, W2, b2r)


# speedup vs baseline: 1.0979x; 1.0008x over previous
import functools

import jax
import jax.numpy as jnp
from jax import lax
from jax.experimental import pallas as pl
from jax.experimental.pallas import tpu as pltpu
from jax.experimental.pallas import tpu_sc as plsc

N = 10000
E = 160000
D_IN = 256
D_HID = 32
N_CLASSES = 128
NUM_GRAPHS = 64

NW = 32
CHUNK = 128
NC_TOT = 80
NC0 = 52
NC1 = NC_TOT - NC0
NCMAX = 64
E_PAD = 16 * NC_TOT * CHUNK
NCHUNKS_ALL = E_PAD // CHUNK
ROWS_PER_TILE = 632
ACC_ROWS = 10112
TAIL_ROWS = N - 15 * ROWS_PER_TILE

NB = 10
BN = 1000

_mesh = plsc.VectorSubcoreMesh(core_axis_name="c", subcore_axis_name="s")


@functools.partial(
    pl.kernel,
    out_type=jax.ShapeDtypeStruct((2, N, 16), jnp.bfloat16),
    mesh=_mesh,
    compiler_params=pltpu.CompilerParams(use_tc_tiling_on_sc=False),
    scratch_types=[
        pltpu.VMEM((NCMAX, CHUNK), jnp.int32),
        pltpu.VMEM((CHUNK, 16), jnp.bfloat16),
        pltpu.VMEM((ROWS_PER_TILE, 16), jnp.bfloat16),
        pltpu.VMEM_SHARED((ACC_ROWS, 16), jnp.bfloat16),
        pltpu.SemaphoreType.DMA,
    ],
)
def _sc_degree(dst_hbm, out_hbm, dst_v, ones_v, zbuf, acc, ssem):
    c = lax.axis_index("c")
    s = lax.axis_index("s")

    z216 = jnp.zeros((2, 16), jnp.bfloat16)
    o216 = jnp.ones((2, 16), jnp.bfloat16)

    def _init(i, carry):
        zbuf[pl.ds(2 * i, 2), :] = z216
        return carry
    lax.fori_loop(0, ROWS_PER_TILE // 2, _init, 0)

    def _ones(i, carry):
        ones_v[pl.ds(2 * i, 2), :] = o216
        return carry
    lax.fori_loop(0, CHUNK // 2, _ones, 0)

    pltpu.sync_copy(zbuf, acc.at[pl.ds(s * ROWS_PER_TILE, ROWS_PER_TILE)])
    plsc.subcore_barrier()

    def _run(nc, base):
        pltpu.sync_copy(dst_hbm.at[pl.ds(base, nc)], dst_v.at[pl.ds(0, nc)])

        def _chunk(j, carry):
            pltpu.async_copy(ones_v, acc.at[dst_v.at[j]], ssem, add=True)
            return carry
        lax.fori_loop(0, nc, _chunk, 0)

        def _drain(j, carry):
            pltpu.make_async_copy(ones_v, acc.at[dst_v.at[0]], ssem).wait()
            return carry
        lax.fori_loop(0, nc, _drain, 0)

    @pl.when(c == 0)
    def _():
        _run(NC0, s * NC0)

    @pl.when(c == 1)
    def _():
        _run(NC1, 16 * NC0 + s * NC1)
    plsc.subcore_barrier()

    @pl.when(s < 15)
    def _():
        pltpu.sync_copy(acc.at[pl.ds(s * ROWS_PER_TILE, ROWS_PER_TILE)],
                        out_hbm.at[c, pl.ds(s * ROWS_PER_TILE, ROWS_PER_TILE)])

    @pl.when(s == 15)
    def _():
        pltpu.sync_copy(acc.at[pl.ds(15 * ROWS_PER_TILE, TAIL_ROWS)],
                        out_hbm.at[c, pl.ds(15 * ROWS_PER_TILE, TAIL_ROWS)])


NBUF = 4
LA = 2

@functools.partial(
    pl.kernel,
    out_type=jax.ShapeDtypeStruct((2, N, D_HID), jnp.bfloat16),
    mesh=_mesh,
    compiler_params=pltpu.CompilerParams(use_tc_tiling_on_sc=False),
    scratch_types=[
        pltpu.VMEM((NCMAX, CHUNK), jnp.int32),
        pltpu.VMEM((NCMAX, CHUNK), jnp.int32),
        pltpu.VMEM((NBUF, CHUNK, D_HID), jnp.bfloat16),
        pltpu.VMEM((ROWS_PER_TILE, D_HID), jnp.bfloat16),
        pltpu.VMEM_SHARED((ACC_ROWS, D_HID), jnp.bfloat16),
        [pltpu.SemaphoreType.DMA] * NBUF,
        [pltpu.SemaphoreType.DMA] * NBUF,
    ],
)
def _sc_aggregate(hs_hbm, src_hbm, dst_hbm, out_hbm,
                  src_v, dst_v, rows_v, zbuf, acc, gsems, ssems):
    c = lax.axis_index("c")
    s = lax.axis_index("s")

    z32 = jnp.zeros((32,), jnp.bfloat16)

    def _init(i, carry):
        zbuf[i, pl.ds(0, 32)] = z32
        return carry
    lax.fori_loop(0, ROWS_PER_TILE, _init, 0)

    pltpu.sync_copy(zbuf, acc.at[pl.ds(s * ROWS_PER_TILE, ROWS_PER_TILE)])
    plsc.subcore_barrier()

    def _gather(jj, b):
        pltpu.async_copy(hs_hbm.at[src_v.at[jj]], rows_v.at[b], gsems[b])

    def _gather_wait(jj, b):
        pltpu.make_async_copy(
            hs_hbm.at[src_v.at[jj]], rows_v.at[b], gsems[b]).wait()

    def _scatter(jj, b):
        pltpu.async_copy(rows_v.at[b], acc.at[dst_v.at[jj]], ssems[b],
                         add=True)

    def _scatter_wait(b):
        pltpu.make_async_copy(rows_v.at[b], acc.at[dst_v.at[0]],
                              ssems[b]).wait()

    def _run(nc, base):
        pltpu.sync_copy(src_hbm.at[pl.ds(base, nc)], src_v.at[pl.ds(0, nc)])
        pltpu.sync_copy(dst_hbm.at[pl.ds(base, nc)], dst_v.at[pl.ds(0, nc)])

        for jj in range(LA):
            _gather(jj, jj % NBUF)

        def _outer(k, carry):
            for b0 in range(NBUF):
                jj = k * NBUF + b0
                bl = (b0 + LA) % NBUF

                @pl.when(jj >= LA)
                def _():
                    _scatter_wait(bl)

                @pl.when(jj + LA < nc)
                def _():
                    _gather(jj + LA, bl)

                _gather_wait(jj, b0)
                _scatter(jj, b0)
            return carry
        lax.fori_loop(0, nc // NBUF, _outer, 0)

        for b in range((nc - LA) % NBUF, (nc - LA) % NBUF + LA):
            _scatter_wait(b % NBUF)

    @pl.when(c == 0)
    def _():
        _run(NC0, s * NC0)

    @pl.when(c == 1)
    def _():
        _run(NC1, 16 * NC0 + s * NC1)
    plsc.subcore_barrier()

    @pl.when(s < 15)
    def _():
        pltpu.sync_copy(acc.at[pl.ds(s * ROWS_PER_TILE, ROWS_PER_TILE)],
                        out_hbm.at[c, pl.ds(s * ROWS_PER_TILE, ROWS_PER_TILE)])

    @pl.when(s == 15)
    def _():
        pltpu.sync_copy(acc.at[pl.ds(15 * ROWS_PER_TILE, TAIL_ROWS)],
                        out_hbm.at[c, pl.ds(15 * ROWS_PER_TILE, TAIL_ROWS)])


def _mm_scale_body(cnt_ref, x_ref, w_ref, hs_ref, dis_ref):
    p = jnp.dot(x_ref[...], w_ref[...], preferred_element_type=jnp.float32)
    deg = (cnt_ref[0, :, 0:1].astype(jnp.float32)
           + cnt_ref[1, :, 0:1].astype(jnp.float32) + 1.0)
    dis = lax.rsqrt(deg)
    dis_ref[...] = dis
    hs_ref[...] = (p * dis).astype(jnp.bfloat16)


def _tc_mm_scale(cnt, x, W1):
    return pl.pallas_call(
        _mm_scale_body,
        grid=(NB,),
        in_specs=[pl.BlockSpec((2, BN, 16), lambda i: (0, i, 0)),
                  pl.BlockSpec((BN, D_IN), lambda i: (i, 0)),
                  pl.BlockSpec((D_IN, D_HID), lambda i: (0, 0))],
        out_specs=[pl.BlockSpec((BN, D_HID), lambda i: (i, 0)),
                   pl.BlockSpec((BN, 1), lambda i: (i, 0))],
        out_shape=[jax.ShapeDtypeStruct((N, D_HID), jnp.bfloat16),
                   jax.ShapeDtypeStruct((N, 1), jnp.float32)],
    )(cnt, x, W1)


def _l1_body(a_ref, hs_ref, dis_ref, b_ref, o_ref):
    ssum = (a_ref[0].astype(jnp.float32) + a_ref[1].astype(jnp.float32)
            + hs_ref[...].astype(jnp.float32))
    h1 = jnp.maximum(ssum * dis_ref[...] + b_ref[...], 0.0)
    o_ref[...] = (h1 * dis_ref[...]).astype(jnp.bfloat16)


def _tc_l1(agg1, hs1, dis, b1r):
    return pl.pallas_call(
        _l1_body,
        grid=(NB,),
        in_specs=[pl.BlockSpec((2, BN, D_HID), lambda i: (0, i, 0)),
                  pl.BlockSpec((BN, D_HID), lambda i: (i, 0)),
                  pl.BlockSpec((BN, 1), lambda i: (i, 0)),
                  pl.BlockSpec((1, D_HID), lambda i: (0, 0))],
        out_specs=pl.BlockSpec((BN, D_HID), lambda i: (i, 0)),
        out_shape=jax.ShapeDtypeStruct((N, D_HID), jnp.bfloat16),
    )(agg1, hs1, dis, b1r)


def _final_body(a_ref, hs_ref, dis_ref, batch_ref, w_ref, b_ref, o_ref,
                sums, cnts):
    i = pl.program_id(0)

    @pl.when(i == 0)
    def _():
        sums[...] = jnp.zeros_like(sums)
        cnts[...] = jnp.zeros_like(cnts)

    m = (a_ref[0].astype(jnp.float32) + a_ref[1].astype(jnp.float32)
         + hs_ref[...].astype(jnp.float32)) * dis_ref[...]
    h2 = jnp.maximum(
        jnp.dot(m, w_ref[...], preferred_element_type=jnp.float32)
        + b_ref[...], 0.0)
    g_iota = lax.broadcasted_iota(jnp.int32, (NUM_GRAPHS, BN), 0)
    bvec = jnp.broadcast_to(batch_ref[0], (NUM_GRAPHS, BN))
    maskf = (g_iota == bvec).astype(jnp.float32)
    sums[...] += jnp.dot(maskf, h2, preferred_element_type=jnp.float32)
    cnts[...] += jnp.broadcast_to(
        jnp.sum(maskf, axis=1, keepdims=True), (NUM_GRAPHS, N_CLASSES))

    @pl.when(i == NB - 1)
    def _():
        pooled = sums[...] / jnp.maximum(cnts[...], 1.0)
        z = pooled - jnp.max(pooled, axis=1, keepdims=True)
        lse = jnp.log(jnp.sum(jnp.exp(z), axis=1, keepdims=True))
        o_ref[...] = z - lse


def _tc_final(agg2, hs2, dis, batch3, W2, b2r):
    return pl.pallas_call(
        _final_body,
        grid=(NB,),
        in_specs=[pl.BlockSpec((2, BN, D_HID), lambda i: (0, i, 0)),
                  pl.BlockSpec((BN, D_HID), lambda i: (i, 0)),
                  pl.BlockSpec((BN, 1), lambda i: (i, 0)),
                  pl.BlockSpec((1, 1, BN), lambda i: (i, 0, 0)),
                  pl.BlockSpec((D_HID, N_CLASSES), lambda i: (0, 0)),
                  pl.BlockSpec((1, N_CLASSES), lambda i: (0, 0))],
        out_specs=pl.BlockSpec((NUM_GRAPHS, N_CLASSES), lambda i: (0, 0)),
        out_shape=jax.ShapeDtypeStruct((NUM_GRAPHS, N_CLASSES), jnp.float32),
        scratch_shapes=[pltpu.VMEM((NUM_GRAPHS, N_CLASSES), jnp.float32),
                        pltpu.VMEM((NUM_GRAPHS, N_CLASSES), jnp.float32)],
    )(agg2, hs2, dis, batch3, W2, b2r)


def kernel(x, edge_index, batch, W1, b1, W2, b2):
    src = edge_index[0]
    dst = edge_index[1]
    pad = E_PAD - E
    src_p = jnp.concatenate(
        [src, jnp.zeros((pad,), jnp.int32)]).reshape(NCHUNKS_ALL, CHUNK)
    dst_p = jnp.concatenate(
        [dst, jnp.full((pad,), N, jnp.int32)]).reshape(NCHUNKS_ALL, CHUNK)
    batch3 = batch.reshape(NB, 1, BN)
    b1r = b1.reshape(1, D_HID)
    b2r = b2.reshape(1, N_CLASSES)

    cnt = _sc_degree(dst_p)
    hs1, dis = _tc_mm_scale(cnt, x, W1)
    agg1 = _sc_aggregate(hs1, src_p, dst_p)
    hs2 = _tc_l1(agg1, hs1, dis, b1r)
    agg2 = _sc_aggregate(hs2, src_p, dst_p)
    return _tc_final(agg2, hs2, dis, batch3, W2, b2r)
